# baseline (speedup 1.0000x reference)
import functools

import jax
import jax.numpy as jnp
from jax import lax
from jax.experimental import pallas as pl
from jax.experimental.pallas import tpu as pltpu
from jax.experimental.pallas import tpu_sc as plsc

DIM = 64
L = 16
CH = 256
IDX_ROWS = CH // 128


def _qembed_body(n_chunks, nc, idx_hbm, wtab_hbm, stab_hbm, out_hbm,
                 idx_v, widx_v, sidx_v, wrows_v, srows_v, orows_v, sem):
    wid = lax.axis_index("s") * nc + lax.axis_index("c")
    base0 = wid * (n_chunks * CH)

    lane = lax.iota(jnp.int32, L)
    byte_shift = (lane & 3) << 3
    spread = lane >> 2

    def chunk_body(ci, carry):
        base = pl.multiple_of(base0 + ci * CH, CH)
        pltpu.sync_copy(idx_hbm.at[pl.ds(base, CH)], idx_v)

        for t in range(CH // L):
            v = idx_v[pl.ds(t * L, L)]
            r, col = divmod(t * L, 128)
            widx_v[r, pl.ds(col, L)] = v >> 4
            sidx_v[r, pl.ds(col, L)] = v >> 6

        copies = []
        for j in range(IDX_ROWS):
            copies.append(pltpu.async_copy(
                wtab_hbm.at[widx_v.at[j]],
                wrows_v.at[pl.ds(j * 128, 128)], sem))
            copies.append(pltpu.async_copy(
                stab_hbm.at[sidx_v.at[j]],
                srows_v.at[pl.ds(j * 128, 128)], sem))
        for cp in copies:
            cp.wait()

        def g_body(g, inner):
            r0 = g * L
            idxv = idx_v[pl.ds(r0, L)]
            for r in range(L):
                iv = idxv[r]
                woff = (iv >> 1) & 7
                w = wrows_v[r0 + r, pl.ds(woff * L, L)]
                soff = (iv >> 5) & 1
                tshift = byte_shift + ((iv & 1) << 2)
                orow = g * (L // 2) + r // 2
                ocol = (r % 2) * DIM
                for v in range(4):
                    shuf = w.at[spread + 4 * v].get(mode="promise_in_bounds")
                    nib = ((shuf >> tshift) & 15) - 8
                    f = nib.astype(jnp.float32)
                    sc = srows_v[r0 + r, pl.ds(soff * DIM + v * L, L)]
                    orows_v[orow, pl.ds(ocol + v * L, L)] = f * sc
            return inner

        lax.fori_loop(0, CH // L, g_body, 0)
        pltpu.sync_copy(
            orows_v,
            out_hbm.at[pl.ds(pl.multiple_of(base // 2, CH // 2), CH // 2)])
        return carry

    lax.fori_loop(0, n_chunks, chunk_body, 0)


def kernel(input, weight, weight_scale):
    n = input.size
    idx_flat = input.reshape(n)
    wtab = lax.bitcast_convert_type(
        weight.reshape(weight.shape[0] // 8, 128, 4), jnp.int32)
    stab = weight_scale.reshape(weight_scale.shape[0] // 2, 2 * DIM)

    mesh = plsc.VectorSubcoreMesh(core_axis_name="c", subcore_axis_name="s")
    nw = mesh.num_cores * mesh.num_subcores
    assert n % (nw * CH) == 0
    n_chunks = n // (nw * CH)

    grid_kernel = pl.kernel(
        functools.partial(_qembed_body, n_chunks, mesh.num_cores),
        out_type=jax.ShapeDtypeStruct((n // 2, 2 * DIM), jnp.float32),
        mesh=mesh,
        scratch_types=[
            pltpu.VMEM((CH,), jnp.int32),
            pltpu.VMEM((IDX_ROWS, 128), jnp.int32),
            pltpu.VMEM((IDX_ROWS, 128), jnp.int32),
            pltpu.VMEM((CH, 128), jnp.int32),
            pltpu.VMEM((CH, 2 * DIM), jnp.float32),
            pltpu.VMEM((CH // 2, 2 * DIM), jnp.float32),
            pltpu.SemaphoreType.DMA,
        ],
    )
    out = grid_kernel(idx_flat, wtab, stab)
    return out.reshape(*input.shape, DIM)


# speedup vs baseline: 3.4294x; 3.4294x over previous
import functools

import jax
import jax.numpy as jnp
from jax import lax
from jax.experimental import pallas as pl
from jax.experimental.pallas import tpu as pltpu
from jax.experimental.pallas import tpu_sc as plsc

DIM = 64
WORDS = DIM // 4
L = 16
CH = 256
IDX_ROWS = CH // 128


def _qembed_body(n_chunks, nc, idx_hbm, wtab_hbm, stab_hbm, out_hbm,
                 idx_v, eidx_v, gidx_v, wrows_v, srows_v, orows_v, sem):
    wid = lax.axis_index("s") * nc + lax.axis_index("c")
    base0 = wid * (n_chunks * CH)

    lane = lax.iota(jnp.int32, L)
    byte_shift = (lane & 3) << 3
    spread = lane >> 2

    def chunk_body(ci, carry):
        base = pl.multiple_of(base0 + ci * CH, CH)
        pltpu.sync_copy(idx_hbm.at[pl.ds(base, CH)], idx_v)

        for t in range(CH // L):
            v = idx_v[pl.ds(t * L, L)]
            r, col = divmod(t * L, 128)
            eidx_v[r, pl.ds(col, L)] = v >> 1
            gidx_v[r, pl.ds(col, L)] = v >> 5

        copies = []
        for j in range(IDX_ROWS):
            copies.append(pltpu.async_copy(
                wtab_hbm.at[eidx_v.at[j]],
                wrows_v.at[pl.ds(j * 128, 128)], sem))
            copies.append(pltpu.async_copy(
                stab_hbm.at[gidx_v.at[j]],
                srows_v.at[pl.ds(j * 128, 128)], sem))
        for cp in copies:
            cp.wait()

        def g_body(g, inner):
            r0 = g * L
            idxv = idx_v[pl.ds(r0, L)]
            for r in range(L):
                iv = idxv[r]
                w = wrows_v[r0 + r]
                tshift = byte_shift + ((iv & 1) << 2)
                for v in range(4):
                    shuf = w.at[spread + 4 * v].get(mode="promise_in_bounds")
                    nib = ((shuf >> tshift) & 15) - 8
                    f = nib.astype(jnp.float32)
                    sc = srows_v[r0 + r, pl.ds(v * L, L)]
                    orows_v[r0 + r, pl.ds(v * L, L)] = f * sc
            return inner

        lax.fori_loop(0, CH // L, g_body, 0)
        pltpu.sync_copy(orows_v, out_hbm.at[pl.ds(base, CH)])
        return carry

    lax.fori_loop(0, n_chunks, chunk_body, 0)


def kernel(input, weight, weight_scale):
    n = input.size
    idx_flat = input.reshape(n)
    z32 = jnp.minimum(idx_flat[0], 0)
    zf = z32.astype(jnp.float32)
    wtab = lax.bitcast_convert_type(
        weight.reshape(weight.shape[0], WORDS, 4), jnp.int32) ^ z32
    stab = weight_scale + zf

    mesh = plsc.VectorSubcoreMesh(core_axis_name="c", subcore_axis_name="s")
    nw = mesh.num_cores * mesh.num_subcores
    assert n % (nw * CH) == 0
    n_chunks = n // (nw * CH)

    grid_kernel = pl.kernel(
        functools.partial(_qembed_body, n_chunks, mesh.num_cores),
        out_type=jax.ShapeDtypeStruct((n, DIM), jnp.float32),
        mesh=mesh,
        scratch_types=[
            pltpu.VMEM((CH,), jnp.int32),
            pltpu.VMEM((IDX_ROWS, 128), jnp.int32),
            pltpu.VMEM((IDX_ROWS, 128), jnp.int32),
            pltpu.VMEM((CH, WORDS), jnp.int32),
            pltpu.VMEM((CH, DIM), jnp.float32),
            pltpu.VMEM((CH, DIM), jnp.float32),
            pltpu.SemaphoreType.DMA,
        ],
        compiler_params=pltpu.CompilerParams(use_tc_tiling_on_sc=False),
    )
    out = grid_kernel(idx_flat, wtab, stab)
    return out.reshape(*input.shape, DIM) + zf


# speedup vs baseline: 4.0446x; 1.1794x over previous
import functools

import jax
import jax.numpy as jnp
from jax import lax
from jax.experimental import pallas as pl
from jax.experimental.pallas import tpu as pltpu
from jax.experimental.pallas import tpu_sc as plsc

DIM = 64
WORDS = DIM // 4
L = 16
BB = 128
HIST_PAD = 8


def _qembed_body(n_h, nc, idx_hbm, wtab_hbm, stab_hbm, out_hbm,
                 tile_v, eidx_v, gidx_v, wrows_v, srows_v, orows_v, sem):
    wid = lax.axis_index("s") * nc + lax.axis_index("c")
    b0 = wid * BB

    lane = lax.iota(jnp.int32, L)
    byte_shift = (lane & 3) << 3
    spread = lane >> 2
    lane128 = lane << 7

    def unit_body(hr, h):
        for t in range(BB // L):
            v = tile_v[hr, pl.ds(t * L, L)]
            eidx_v[0, pl.ds(t * L, L)] = v >> 1
            gidx_v[0, pl.ds(t * L, L)] = v >> 5
        pltpu.async_copy(wtab_hbm.at[eidx_v.at[0]], wrows_v, sem)
        pltpu.async_copy(stab_hbm.at[gidx_v.at[0]], srows_v, sem).wait()
        pltpu.make_async_copy(wtab_hbm.at[eidx_v.at[0]], wrows_v, sem).wait()

        def g_body(g, inner):
            r0 = g * L
            idxv = tile_v[hr, pl.ds(r0, L)]
            for r in range(L):
                iv = idxv[r]
                w = wrows_v[r0 + r]
                tshift = byte_shift + ((iv & 1) << 2)
                for v in range(4):
                    shuf = w.at[spread + 4 * v].get(mode="promise_in_bounds")
                    nib = ((shuf >> tshift) & 15) - 8
                    f = nib.astype(jnp.float32)
                    sc = srows_v[r0 + r, pl.ds(v * L, L)]
                    pos = lane128 + (v * L * BB + r0 + r)
                    plsc.store_scatter(orows_v, [pos], f * sc)
            return inner

        lax.fori_loop(0, BB // L, g_body, 0)

        ocopies = [
            pltpu.async_copy(
                orows_v.at[pl.ds(dblk * 8 * BB, 8 * BB)],
                out_hbm.at[h, dblk, wid], sem)
            for dblk in range(8)
        ]
        for cp in ocopies:
            cp.wait()

    def hblk_body(hblk, carry):
        h0 = hblk * HIST_PAD
        pltpu.sync_copy(
            idx_hbm.at[pl.ds(h0, HIST_PAD), pl.ds(b0, BB)], tile_v)

        def hr_body(hr, inner):
            unit_body(hr, h0 + hr)
            return inner

        lax.fori_loop(0, HIST_PAD, hr_body, 0)
        return carry

    lax.fori_loop(0, n_h // HIST_PAD, hblk_body, 0)

    tail = n_h % HIST_PAD
    if tail:
        h0 = n_h - tail
        pltpu.sync_copy(
            idx_hbm.at[pl.ds(h0, tail), pl.ds(b0, BB)],
            tile_v.at[pl.ds(0, tail)])

        def tail_body(hr, inner):
            unit_body(hr, h0 + hr)
            return inner

        lax.fori_loop(0, tail, tail_body, 0)


def kernel(input, weight, weight_scale):
    nb, n_h = input.shape
    idx_t = input.T
    wtab = lax.bitcast_convert_type(
        weight.reshape(-1).reshape(weight.shape[0], WORDS, 4), jnp.int32)
    stab = weight_scale.reshape(-1).reshape(weight_scale.shape)

    mesh = plsc.VectorSubcoreMesh(core_axis_name="c", subcore_axis_name="s")
    nw = mesh.num_cores * mesh.num_subcores
    assert nb % (nw * BB) == 0 and nb // BB == nw

    grid_kernel = pl.kernel(
        functools.partial(_qembed_body, n_h, mesh.num_cores),
        out_type=jax.ShapeDtypeStruct((n_h, DIM // 8, nb // BB, 8 * BB),
                                      jnp.float32),
        mesh=mesh,
        scratch_types=[
            pltpu.VMEM((HIST_PAD, BB), jnp.int32),
            pltpu.VMEM((1, BB), jnp.int32),
            pltpu.VMEM((1, BB), jnp.int32),
            pltpu.VMEM((BB, WORDS), jnp.int32),
            pltpu.VMEM((BB, DIM), jnp.float32),
            pltpu.VMEM((DIM * BB,), jnp.float32),
            pltpu.SemaphoreType.DMA,
        ],
        compiler_params=pltpu.CompilerParams(use_tc_tiling_on_sc=False,
                                             needs_layout_passes=False),
    )
    out4 = grid_kernel(idx_t, wtab, stab)
    out5 = out4.reshape(n_h, DIM // 8, nb // BB, 8, BB)
    return out5.transpose(2, 4, 0, 1, 3).reshape(nb, n_h, DIM)


# speedup vs baseline: 4.6666x; 1.1538x over previous
import functools

import jax
import jax.numpy as jnp
from jax import lax
from jax.experimental import pallas as pl
from jax.experimental.pallas import tpu as pltpu
from jax.experimental.pallas import tpu_sc as plsc

DIM = 64
WORDS = DIM // 4
L = 16
BB = 128
HIST_PAD = 8


def _qembed_body(n_h, nc, idx_hbm, wtab_hbm, stab_hbm, out_hbm,
                 tile_v, eidx_v, gidx_v, wrows_v, srows_v, orows_v, sem):
    wid = lax.axis_index("s") * nc + lax.axis_index("c")
    b0 = wid * BB

    lane = lax.iota(jnp.int32, L)
    byte_shift = (lane & 3) << 3
    spread = lane >> 2

    def unit_body(hr, h):
        for t in range(BB // L):
            v = tile_v[hr, pl.ds(t * L, L)]
            eidx_v[0, pl.ds(t * L, L)] = v >> 1
            gidx_v[0, pl.ds(t * L, L)] = v >> 5
        pltpu.async_copy(wtab_hbm.at[eidx_v.at[0]], wrows_v, sem)
        pltpu.async_copy(stab_hbm.at[gidx_v.at[0]], srows_v, sem).wait()
        pltpu.make_async_copy(wtab_hbm.at[eidx_v.at[0]], wrows_v, sem).wait()

        def g_body(g, inner):
            r0 = g * L
            idxv = tile_v[hr, pl.ds(r0, L)]
            for r in range(L):
                iv = idxv[r]
                w = wrows_v[r0 + r]
                tshift = byte_shift + ((iv & 1) << 2)
                bc = jnp.full((L,), r0 + r, jnp.int32)
                for v in range(4):
                    shuf = w.at[spread + 4 * v].get(mode="promise_in_bounds")
                    nib = ((shuf >> tshift) & 15) - 8
                    f = nib.astype(jnp.float32)
                    sc = srows_v[r0 + r, pl.ds(v * L, L)]
                    plsc.store_scatter(orows_v, [v * L + lane, bc], f * sc)
            return inner

        lax.fori_loop(0, BB // L, g_body, 0)

        ocopies = [
            pltpu.async_copy(
                orows_v.at[pl.ds(dblk * 8, 8), pl.ds(0, BB)],
                out_hbm.at[h, dblk, wid], sem)
            for dblk in range(8)
        ]
        for cp in ocopies:
            cp.wait()

    def hblk_body(hblk, carry):
        h0 = hblk * HIST_PAD
        pltpu.sync_copy(
            idx_hbm.at[pl.ds(h0, HIST_PAD), pl.ds(b0, BB)], tile_v)

        def hr_body(hr, inner):
            unit_body(hr, h0 + hr)
            return inner

        lax.fori_loop(0, HIST_PAD, hr_body, 0)
        return carry

    lax.fori_loop(0, n_h // HIST_PAD, hblk_body, 0)

    tail = n_h % HIST_PAD
    if tail:
        h0 = n_h - tail
        pltpu.sync_copy(
            idx_hbm.at[pl.ds(h0, tail), pl.ds(b0, BB)],
            tile_v.at[pl.ds(0, tail)])

        def tail_body(hr, inner):
            unit_body(hr, h0 + hr)
            return inner

        lax.fori_loop(0, tail, tail_body, 0)


def kernel(input, weight, weight_scale):
    nb, n_h = input.shape
    idx_t = input.T
    wtab = lax.bitcast_convert_type(
        weight.reshape(-1).reshape(weight.shape[0], WORDS, 4), jnp.int32)
    stab = weight_scale.reshape(-1).reshape(weight_scale.shape)

    mesh = plsc.VectorSubcoreMesh(core_axis_name="c", subcore_axis_name="s")
    nw = mesh.num_cores * mesh.num_subcores
    assert nb % (nw * BB) == 0 and nb // BB == nw

    grid_kernel = pl.kernel(
        functools.partial(_qembed_body, n_h, mesh.num_cores),
        out_type=jax.ShapeDtypeStruct((n_h, DIM // 8, nb // BB, 8, BB),
                                      jnp.float32),
        mesh=mesh,
        scratch_types=[
            pltpu.VMEM((HIST_PAD, BB), jnp.int32),
            pltpu.VMEM((1, BB), jnp.int32),
            pltpu.VMEM((1, BB), jnp.int32),
            pltpu.VMEM((BB, WORDS), jnp.int32),
            pltpu.VMEM((BB, DIM), jnp.float32),
            pltpu.VMEM((DIM, BB + 5), jnp.float32),
            pltpu.SemaphoreType.DMA,
        ],
        compiler_params=pltpu.CompilerParams(use_tc_tiling_on_sc=False,
                                             needs_layout_passes=False),
    )
    out5 = grid_kernel(idx_t, wtab, stab)
    return out5.transpose(2, 4, 0, 1, 3).reshape(nb, n_h, DIM)


# speedup vs baseline: 8.0550x; 1.7261x over previous
import functools

import jax
import jax.numpy as jnp
from jax import lax
from jax.experimental import pallas as pl
from jax.experimental.pallas import tpu as pltpu
from jax.experimental.pallas import tpu_sc as plsc

DIM = 64
WORDS = DIM // 4
L = 16
BB = 128
HIST_PAD = 8


def _qembed_body(n_h, nc, idx_hbm, wtab_hbm, stab_hbm, out_hbm,
                 tile_v, eidx_v, gidx_v, wrows_v, srows_v, orows_v, sem):
    wid = lax.axis_index("s") * nc + lax.axis_index("c")
    b0 = wid * BB

    lane = lax.iota(jnp.int32, L)
    byte_shift = (lane & 3) << 3
    spread = lane >> 2

    def unit_body(hr, h):
        for t in range(BB // L):
            v = tile_v[hr, pl.ds(t * L, L)]
            eidx_v[0, pl.ds(t * L, L)] = v >> 1
            gidx_v[0, pl.ds(t * L, L)] = v >> 5
        pltpu.async_copy(wtab_hbm.at[eidx_v.at[0]], wrows_v, sem)
        pltpu.async_copy(stab_hbm.at[gidx_v.at[0]], srows_v, sem).wait()
        pltpu.make_async_copy(wtab_hbm.at[eidx_v.at[0]], wrows_v, sem).wait()

        def g_body(g, inner):
            r0 = g * L
            idxv = tile_v[hr, pl.ds(r0, L)]
            for r in range(L):
                iv = idxv[r]
                w = wrows_v[r0 + r]
                tshift = byte_shift + ((iv & 1) << 2)
                bc = jnp.full((L,), r0 + r, jnp.int32)
                for v in range(4):
                    shuf = w.at[spread + 4 * v].get(mode="promise_in_bounds")
                    nib = ((shuf >> tshift) & 15) - 8
                    f = nib.astype(jnp.float32)
                    sc = srows_v[r0 + r, pl.ds(v * L, L)]
                    plsc.store_scatter(orows_v, [v * L + lane, bc], f * sc)
            return inner

        lax.fori_loop(0, BB // L, g_body, 0)

        ocopies = [
            pltpu.async_copy(
                orows_v.at[pl.ds(dblk * 8, 8), pl.ds(0, BB)],
                out_hbm.at[h, dblk, wid], sem)
            for dblk in range(8)
        ]
        for cp in ocopies:
            cp.wait()

    def hblk_body(hblk, carry):
        h0 = hblk * HIST_PAD
        pltpu.sync_copy(
            idx_hbm.at[pl.ds(h0, HIST_PAD), pl.ds(b0, BB)], tile_v)

        def hr_body(hr, inner):
            unit_body(hr, h0 + hr)
            return inner

        lax.fori_loop(0, HIST_PAD, hr_body, 0)
        return carry

    lax.fori_loop(0, n_h // HIST_PAD, hblk_body, 0)

    tail = n_h % HIST_PAD
    if tail:
        h0 = n_h - tail
        pltpu.sync_copy(
            idx_hbm.at[pl.ds(h0, tail), pl.ds(b0, BB)],
            tile_v.at[pl.ds(0, tail)])

        def tail_body(hr, inner):
            unit_body(hr, h0 + hr)
            return inner

        lax.fori_loop(0, tail, tail_body, 0)


import numpy as np

_C = np.arange(512)[:, None]
_CP = np.arange(128)[None, :]
_MLO_NP = ((_C // 4 == _CP) * np.where(_C % 4 == 0, 1, 0)
           + (_C // 4 == _CP) * np.where(_C % 4 == 1, 256, 0)
           ).astype(np.float32)
_MHI_NP = ((_C // 4 == _CP) * np.where(_C % 4 == 2, 1, 0)
           + (_C // 4 == _CP) * np.where(_C % 4 == 3, 256, 0)
           ).astype(np.float32)


def _repack_w_body(w_ref, mlo_ref, mhi_ref, wtab_ref):
    x = w_ref[0].astype(jnp.bfloat16)
    zlo = jnp.dot(x, mlo_ref[...], preferred_element_type=jnp.float32)
    zhi = jnp.dot(x, mhi_ref[...], preferred_element_type=jnp.float32)
    wtab_ref[0] = zlo.astype(jnp.int32) | (zhi.astype(jnp.int32) << 16)


def _repack_s_body(s_ref, stab_ref):
    stab_ref[...] = s_ref[...]


def _repack(weight, weight_scale):
    nw = weight.shape[0]
    grid = 125
    bw = nw * DIM // 512 // grid
    w4 = weight.reshape(grid, bw, 512)
    wtab2 = pl.pallas_call(
        _repack_w_body,
        grid=(grid,),
        in_specs=[
            pl.BlockSpec((1, bw, 512), lambda i: (i, 0, 0)),
            pl.BlockSpec((512, 2 * DIM), lambda i: (0, 0)),
            pl.BlockSpec((512, 2 * DIM), lambda i: (0, 0)),
        ],
        out_specs=pl.BlockSpec((1, bw, 2 * DIM), lambda i: (i, 0, 0)),
        out_shape=jax.ShapeDtypeStruct((grid, bw, 2 * DIM), jnp.int32),
    )(w4, jnp.asarray(_MLO_NP, dtype=jnp.bfloat16),
      jnp.asarray(_MHI_NP, dtype=jnp.bfloat16))
    ns = weight_scale.shape[0]
    bs = ns // 2 // grid
    s3 = weight_scale.reshape(grid, bs, 2 * DIM)
    stab2 = pl.pallas_call(
        _repack_s_body,
        grid=(grid,),
        in_specs=[pl.BlockSpec((1, bs, 2 * DIM), lambda i: (i, 0, 0))],
        out_specs=pl.BlockSpec((1, bs, 2 * DIM), lambda i: (i, 0, 0)),
        out_shape=jax.ShapeDtypeStruct((grid, bs, 2 * DIM), jnp.float32),
    )(s3)
    return (wtab2.reshape(nw, WORDS), stab2.reshape(weight_scale.shape))


def kernel(input, weight, weight_scale):
    nb, n_h = input.shape
    idx_t = input.T
    wtab, stab = _repack(weight, weight_scale)

    mesh = plsc.VectorSubcoreMesh(core_axis_name="c", subcore_axis_name="s")
    nw = mesh.num_cores * mesh.num_subcores
    assert nb % (nw * BB) == 0 and nb // BB == nw

    grid_kernel = pl.kernel(
        functools.partial(_qembed_body, n_h, mesh.num_cores),
        out_type=jax.ShapeDtypeStruct((n_h, DIM // 8, nb // BB, 8, BB),
                                      jnp.float32),
        mesh=mesh,
        scratch_types=[
            pltpu.VMEM((HIST_PAD, BB), jnp.int32),
            pltpu.VMEM((1, BB), jnp.int32),
            pltpu.VMEM((1, BB), jnp.int32),
            pltpu.VMEM((BB, WORDS), jnp.int32),
            pltpu.VMEM((BB, DIM), jnp.float32),
            pltpu.VMEM((DIM, BB + 5), jnp.float32),
            pltpu.SemaphoreType.DMA,
        ],
        compiler_params=pltpu.CompilerParams(use_tc_tiling_on_sc=False,
                                             needs_layout_passes=False),
    )
    out5 = grid_kernel(idx_t, wtab, stab)
    return out5.transpose(2, 4, 0, 1, 3).reshape(nb, n_h, DIM)
